# baseline (speedup 1.0000x reference)
import jax
import jax.numpy as jnp
from jax import lax
from jax.experimental import pallas as pl
from jax.experimental.pallas import tpu as pltpu
from jax.experimental.pallas import tpu_sc as plsc

B = 4096
L = 200
D = 32
TOK = 1000000
NC = 2
NS = 16
NW = NC * NS
LANES = 16

NBLK = TOK // 128 + 1
SLOTS = (NBLK + NW - 1) // NW
TAIL_T0 = (TOK // 128) * 128


def _tr_body(tt_hbm, tail_hbm, lin_hbm, in_v0, in_v1, out_v0, out_v1,
             semi0, semi1, semo0, semo1):
    w = lax.axis_index("s") * NC + lax.axis_index("c")
    col32 = lax.iota(jnp.int32, LANES) * D

    def t0_of(slot):
        return pl.multiple_of((slot * NW + w) * 128, 128)

    def fire_in(slot, in_v, semi):
        t0 = t0_of(slot)

        @pl.when(t0 + 128 <= TOK)
        def _():
            pltpu.async_copy(tt_hbm.at[:, pl.ds(t0, 128)], in_v, semi)

        @pl.when(t0 == TAIL_T0)
        def _():
            pltpu.async_copy(tail_hbm, in_v, semi)

    def process(slot, in_v, out_v, semi, semo):
        t0 = t0_of(slot)
        is_tail = t0 == TAIL_T0

        @pl.when(jnp.logical_or(t0 + 128 <= TOK, is_tail))
        def _():
            pltpu.make_async_copy(tt_hbm.at[:, pl.ds(0, 128)], in_v,
                                  semi).wait()

            @pl.when(slot >= 2)
            def _():
                pltpu.make_async_copy(lin_hbm.at[pl.ds(0, 128 * D)],
                                      out_v, semo).wait()

            @pl.loop(0, D)
            def _(d):
                for g in range(8):
                    vals = in_v[d, pl.ds(g * LANES, LANES)]
                    plsc.store_scatter(out_v,
                                       [col32 + (g * LANES * D + d)], vals)

            @pl.when(t0 + 128 <= TOK)
            def _():
                pltpu.async_copy(out_v, lin_hbm.at[pl.ds(t0 * D, 128 * D)],
                                 semo)

            @pl.when(is_tail)
            def _():
                pltpu.async_copy(out_v.at[pl.ds(64 * D, 64 * D)],
                                 lin_hbm.at[pl.ds(t0 * D, 64 * D)], semo)

    fire_in(0, in_v0, semi0)

    @pl.loop(0, (SLOTS + 1) // 2)
    def _(k):
        s = k * 2
        fire_in(s + 1, in_v1, semi1)
        process(s, in_v0, out_v0, semi0, semo0)
        fire_in(s + 2, in_v0, semi0)
        process(s + 1, in_v1, out_v1, semi1, semo1)

    for last, out_v, semo in ((SLOTS - 2, out_v1, semo1),
                              (SLOTS - 1, out_v0, semo0)):
        t0 = t0_of(last)

        @pl.when(t0 + 128 <= TOK)
        def _():
            pltpu.make_async_copy(lin_hbm.at[pl.ds(0, 128 * D)],
                                  out_v, semo).wait()

        @pl.when(t0 == TAIL_T0)
        def _():
            pltpu.make_async_copy(lin_hbm.at[pl.ds(0, 64 * D)],
                                  out_v.at[pl.ds(0, 64 * D)], semo).wait()


BPW = B // NW
UL = 4
LC = L // (2 * UL)
OUTN = B * L * D


def _gat_body(xt_hbm, lin_hbm, pos_hbm, out_hbm, idx_v, rows_v, obuf_v,
              pos_v, semg, semo):
    w = lax.axis_index("s") * NC + lax.axis_index("c")
    b0 = w * BPW
    iota128 = lax.iota(jnp.int32, LANES) * BPW

    pltpu.sync_copy(pos_hbm, pos_v)

    @pl.loop(0, LC)
    def _(lc):
        l0 = lc * 2 * UL
        pltpu.sync_copy(xt_hbm.at[pl.ds(l0, 2 * UL), pl.ds(b0, BPW)], idx_v)

        for half in range(2):
            @pl.when(lc > 0)
            def _():
                pltpu.make_async_copy(out_hbm.at[pl.ds(0, UL * D * BPW)],
                                      obuf_v.at[half], semo.at[half]).wait()

            for li in range(UL):
                pltpu.async_copy(lin_hbm.at[idx_v.at[half * UL + li]],
                                 rows_v.at[half, li], semg.at[half])

        for half in range(2):
            for li in range(UL):
                pltpu.make_async_copy(lin_hbm.at[pl.ds(0, BPW)],
                                      rows_v.at[half, li],
                                      semg.at[half]).wait()

            for li in range(UL):
                lpos = l0 + half * UL + li
                p0 = pos_v[pl.ds(lpos * D, LANES)]
                p1 = pos_v[pl.ds(lpos * D + LANES, LANES)]

                @pl.loop(0, BPW)
                def _(bi):
                    v0 = rows_v[half, li, bi, pl.ds(0, LANES)] * p0
                    v1 = rows_v[half, li, bi, pl.ds(LANES, LANES)] * p1
                    base = li * (D * BPW) + bi
                    plsc.store_scatter(obuf_v.at[half], [iota128 + base], v0)
                    plsc.store_scatter(obuf_v.at[half],
                                       [iota128 + (base + LANES * BPW)], v1)

            for li in range(UL):
                lpos = l0 + half * UL + li
                for di in range(4):
                    off = ((lpos * 4 + di) * 32 + w) * 1024
                    pltpu.async_copy(
                        obuf_v.at[half, pl.ds((li * 4 + di) * 1024, 1024)],
                        out_hbm.at[pl.ds(off, 1024)], semo.at[half])

    @pl.loop(0, 2)
    def _(half):
        pltpu.make_async_copy(out_hbm.at[pl.ds(0, UL * D * BPW)],
                              obuf_v.at[half], semo.at[half]).wait()


_MESH = plsc.VectorSubcoreMesh(core_axis_name="c", subcore_axis_name="s")


@jax.jit
def _encode(x, token_table, pos_table):
    tt_t = jnp.transpose(token_table)
    tt_tail = lax.slice(tt_t, (0, TOK - 128), (D, TOK))
    xt = jnp.transpose(x)
    pos_lin = pos_table.reshape(L * D)

    tr = pl.kernel(
        _tr_body,
        out_type=jax.ShapeDtypeStruct((TOK * D,), jnp.float32),
        mesh=_MESH,
        compiler_params=pltpu.CompilerParams(use_tc_tiling_on_sc=True,
                                             needs_layout_passes=False),
        scratch_types=[
            pltpu.VMEM((D, 128), jnp.float32),
            pltpu.VMEM((D, 128), jnp.float32),
            pltpu.VMEM((128 * D,), jnp.float32),
            pltpu.VMEM((128 * D,), jnp.float32),
            pltpu.SemaphoreType.DMA,
            pltpu.SemaphoreType.DMA,
            pltpu.SemaphoreType.DMA,
            pltpu.SemaphoreType.DMA,
        ],
    )
    lin = tr(tt_t, tt_tail)

    gat = pl.kernel(
        _gat_body,
        out_type=jax.ShapeDtypeStruct((OUTN,), jnp.float32),
        mesh=_MESH,
        compiler_params=pltpu.CompilerParams(use_tc_tiling_on_sc=False,
                                             needs_layout_passes=False),
        scratch_types=[
            pltpu.VMEM((2 * UL, BPW), jnp.int32),
            pltpu.VMEM((2, UL, BPW, D), jnp.float32),
            pltpu.VMEM((2, UL * D * BPW), jnp.float32),
            pltpu.VMEM((L * D,), jnp.float32),
            pltpu.SemaphoreType.DMA((2,)),
            pltpu.SemaphoreType.DMA((2,)),
        ],
    )
    out_flat = gat(xt, lin.reshape(TOK, D), pos_lin)

    out5 = out_flat.reshape(L, 4, 32, 8, BPW)
    return jnp.transpose(out5, (2, 4, 0, 1, 3)).reshape(B, L, D)


def kernel(x, token_table, pos_table):
    return _encode(x.astype(jnp.int32), token_table, pos_table)
